# Initial kernel scaffold; baseline (speedup 1.0000x reference)
import jax
import jax.numpy as jnp
from jax.experimental import pallas as pl


def kernel(x, edge_index, W1, b1, W2, b2):
    raise NotImplementedError("write your pallas kernel here")


# speedup vs baseline: 22.6323x; 22.6323x over previous
import functools

import jax
import jax.numpy as jnp
from jax import lax
from jax.experimental import pallas as pl
from jax.experimental.pallas import tpu as pltpu
from jax.experimental.pallas import tpu_sc as plsc

N = 10000
D = 128
HID = 256
N_PAD = 10240
NC = 2
NS = 16
NW = NC * NS
CHUNK = 128
E = 320000
K = (-(-E // (NW * CHUNK)) + 7) // 8 * 8
EP = NW * K * CHUNK
RPT = N_PAD // NS

_mesh = plsc.VectorSubcoreMesh(core_axis_name="c", subcore_axis_name="s")


def _deg_body(zeros_hbm, dst_hbm, out_hbm, dst_v, ones_v, acc):
    c = lax.axis_index("c")
    s = lax.axis_index("s")
    wid = c * NS + s
    sl = pl.ds(s * RPT, RPT)

    def fill_ones(i, carry):
        for col in range(D // 16):
            ones_v[i, pl.ds(col * 16, 16)] = jnp.ones((16,), jnp.float32)
        return carry
    lax.fori_loop(0, CHUNK, fill_ones, 0)

    pltpu.sync_copy(zeros_hbm.at[sl], acc.at[sl])
    pltpu.sync_copy(dst_hbm.at[pl.ds(wid * K, K)], dst_v)
    plsc.subcore_barrier()

    def chunk(j, carry):
        pltpu.sync_copy(ones_v, acc.at[dst_v.at[j]], add=True)
        return carry
    lax.fori_loop(0, K, chunk, 0)

    plsc.subcore_barrier()
    pltpu.sync_copy(acc.at[sl], out_hbm.at[c, sl])


_deg_kernel = functools.partial(
    pl.kernel,
    out_type=jax.ShapeDtypeStruct((NC, N_PAD, D), jnp.float32),
    mesh=_mesh,
    scratch_types=[
        pltpu.VMEM((K, CHUNK), jnp.int32),
        pltpu.VMEM((CHUNK, D), jnp.float32),
        pltpu.VMEM_SHARED((N_PAD, D), jnp.float32),
    ],
)(_deg_body)


def _scatter_body(xs_hbm, zeros_hbm, src_hbm, dst_hbm, out_hbm,
                  src_v, dst_v, rows_v, acc, sem):
    c = lax.axis_index("c")
    s = lax.axis_index("s")
    wid = c * NS + s
    sl = pl.ds(s * RPT, RPT)

    @pl.when(c == 0)
    def _():
        pltpu.sync_copy(xs_hbm.at[sl], acc.at[sl])

    @pl.when(c == 1)
    def _():
        pltpu.sync_copy(zeros_hbm.at[sl], acc.at[sl])

    pltpu.sync_copy(src_hbm.at[pl.ds(wid * K, K)], src_v)
    pltpu.sync_copy(dst_hbm.at[pl.ds(wid * K, K)], dst_v)
    plsc.subcore_barrier()

    def chunk(j, carry):
        pltpu.async_copy(xs_hbm.at[src_v.at[j]], rows_v, sem).wait()
        pltpu.sync_copy(rows_v, acc.at[dst_v.at[j]], add=True)
        return carry
    lax.fori_loop(0, K, chunk, 0)

    plsc.subcore_barrier()
    pltpu.sync_copy(acc.at[sl], out_hbm.at[c, sl])


_scatter_kernel = functools.partial(
    pl.kernel,
    out_type=jax.ShapeDtypeStruct((NC, N_PAD, D), jnp.float32),
    mesh=_mesh,
    scratch_types=[
        pltpu.VMEM((K, CHUNK), jnp.int32),
        pltpu.VMEM((K, CHUNK), jnp.int32),
        pltpu.VMEM((CHUNK, D), jnp.float32),
        pltpu.VMEM_SHARED((N_PAD, D), jnp.float32),
        pltpu.SemaphoreType.DMA,
    ],
)(_scatter_body)


def _prescale_body(x_ref, deg_ref, xs_ref, dinv_ref):
    w = deg_ref[0] + deg_ref[1]
    cnt = jnp.sum(w, axis=1, keepdims=True) * (1.0 / D)
    dinv = lax.rsqrt(cnt + 1.0)
    dinv_ref[...] = dinv
    xs_ref[...] = x_ref[...] * dinv


def _mid_body(p_ref, dinv_ref, w1_ref, b1_ref, w2_ref, t_ref):
    dinv = dinv_ref[...]
    agg = (p_ref[0] + p_ref[1]) * dinv
    h = jnp.dot(agg, w1_ref[...], preferred_element_type=jnp.float32)
    h = jnp.maximum(h + b1_ref[...], 0.0)
    t_ref[...] = jnp.dot(h, w2_ref[...],
                         preferred_element_type=jnp.float32) * dinv


def _final_body(q_ref, dinv_ref, b2_ref, o_ref):
    o_ref[...] = ((q_ref[0, :N] + q_ref[1, :N]) * dinv_ref[:N]
                  + b2_ref[...])


@jax.jit
def kernel(x, edge_index, W1, b1, W2, b2):
    src = edge_index[0].astype(jnp.int32)
    dst = edge_index[1].astype(jnp.int32)
    pad = N + (jnp.arange(EP - E, dtype=jnp.int32) % (N_PAD - N))
    src_p = jnp.concatenate([src, pad]).reshape(NW * K, CHUNK)
    dst_p = jnp.concatenate([dst, pad]).reshape(NW * K, CHUNK)

    x_pad = jnp.concatenate([x, jnp.zeros((N_PAD - N, D), x.dtype)])
    zeros = jnp.zeros((N_PAD, D), jnp.float32)

    deg = _deg_kernel(zeros, dst_p)

    xs, dinv = pl.pallas_call(
        _prescale_body,
        out_shape=[
            jax.ShapeDtypeStruct((N_PAD, D), jnp.float32),
            jax.ShapeDtypeStruct((N_PAD, 1), jnp.float32),
        ],
    )(x_pad, deg)

    p = _scatter_kernel(xs, zeros, src_p, dst_p)

    t = pl.pallas_call(
        _mid_body,
        out_shape=jax.ShapeDtypeStruct((N_PAD, D), jnp.float32),
    )(p, dinv, W1, b1.reshape(1, HID), W2)

    q = _scatter_kernel(t, zeros, src_p, dst_p)

    out = pl.pallas_call(
        _final_body,
        out_shape=jax.ShapeDtypeStruct((N, D), jnp.float32),
    )(q, dinv, b2.reshape(1, D))
    return out


# speedup vs baseline: 25.2050x; 1.1137x over previous
import functools

import jax
import jax.numpy as jnp
from jax import lax
from jax.experimental import pallas as pl
from jax.experimental.pallas import tpu as pltpu
from jax.experimental.pallas import tpu_sc as plsc

N = 10000
D = 128
HID = 256
N_PAD = 10240
NC = 2
NS = 16
NW = NC * NS
CHUNK = 128
E = 320000
K = (-(-E // (NW * CHUNK)) + 7) // 8 * 8
EP = NW * K * CHUNK
RPT = N_PAD // NS
IB = 16

_mesh = plsc.VectorSubcoreMesh(core_axis_name="c", subcore_axis_name="s")


def _deg_body(zeros_hbm, dst_hbm, out_hbm, dst_v, ones_v, acc, sem):
    c = lax.axis_index("c")
    s = lax.axis_index("s")
    wid = c * NS + s
    sl = pl.ds(s * RPT, RPT)

    def fill_ones(i, carry):
        for col in range(D // 16):
            ones_v[i, pl.ds(col * 16, 16)] = jnp.ones((16,), jnp.float32)
        return carry
    lax.fori_loop(0, CHUNK, fill_ones, 0)

    pltpu.sync_copy(zeros_hbm.at[sl], acc.at[sl])
    pltpu.sync_copy(dst_hbm.at[pl.ds(wid * K, K)], dst_v)
    plsc.subcore_barrier()

    FIRE = 8

    def batch(b, carry):
        for u in range(FIRE):
            pltpu.async_copy(ones_v, acc.at[dst_v.at[b * FIRE + u]],
                             sem, add=True)
        for u in range(FIRE):
            pltpu.make_async_copy(ones_v, acc.at[dst_v.at[b * FIRE + u]],
                                  sem).wait()
        return carry
    lax.fori_loop(0, K // FIRE, batch, 0)

    plsc.subcore_barrier()
    pltpu.sync_copy(acc.at[sl], out_hbm.at[c, sl])


_deg_kernel = functools.partial(
    pl.kernel,
    out_type=jax.ShapeDtypeStruct((NC, N_PAD, D), jnp.float32),
    mesh=_mesh,
    scratch_types=[
        pltpu.VMEM((K, CHUNK), jnp.int32),
        pltpu.VMEM((CHUNK, D), jnp.float32),
        pltpu.VMEM_SHARED((N_PAD, D), jnp.float32),
        pltpu.SemaphoreType.DMA,
    ],
)(_deg_body)


def _scatter_body(xs_hbm, zeros_hbm, src_hbm, dst_hbm, out_hbm,
                  src_v, dst_v, rows_a, rows_b, acc, gs_a, gs_b, ss_a, ss_b):
    c = lax.axis_index("c")
    s = lax.axis_index("s")
    wid = c * NS + s
    sl = pl.ds(s * RPT, RPT)

    @pl.when(c == 0)
    def _():
        pltpu.sync_copy(xs_hbm.at[sl], acc.at[sl])

    @pl.when(c == 1)
    def _():
        pltpu.sync_copy(zeros_hbm.at[sl], acc.at[sl])

    plsc.subcore_barrier()

    def stage(g, carry):
        base = wid * K + g * IB
        pltpu.sync_copy(src_hbm.at[pl.ds(base, IB)], src_v)
        pltpu.sync_copy(dst_hbm.at[pl.ds(base, IB)], dst_v)
        pltpu.async_copy(xs_hbm.at[src_v.at[0]], rows_a, gs_a)
        pltpu.async_copy(xs_hbm.at[src_v.at[1]], rows_b, gs_b)

        def pair(i, c2):
            j = 2 * i
            pltpu.make_async_copy(xs_hbm.at[src_v.at[j]], rows_a, gs_a).wait()
            pltpu.async_copy(rows_a, acc.at[dst_v.at[j]], ss_a, add=True)
            pltpu.make_async_copy(xs_hbm.at[src_v.at[j + 1]], rows_b,
                                  gs_b).wait()
            pltpu.async_copy(rows_b, acc.at[dst_v.at[j + 1]], ss_b, add=True)

            @pl.when(j + 2 < IB)
            def _():
                pltpu.make_async_copy(rows_a, acc.at[dst_v.at[j]],
                                      ss_a).wait()
                pltpu.async_copy(xs_hbm.at[src_v.at[j + 2]], rows_a, gs_a)

            @pl.when(j + 3 < IB)
            def _():
                pltpu.make_async_copy(rows_b, acc.at[dst_v.at[j + 1]],
                                      ss_b).wait()
                pltpu.async_copy(xs_hbm.at[src_v.at[j + 3]], rows_b, gs_b)
            return c2
        lax.fori_loop(0, IB // 2, pair, 0)
        pltpu.make_async_copy(rows_a, acc.at[dst_v.at[IB - 2]], ss_a).wait()
        pltpu.make_async_copy(rows_b, acc.at[dst_v.at[IB - 1]], ss_b).wait()
        return carry
    lax.fori_loop(0, K // IB, stage, 0)

    plsc.subcore_barrier()
    pltpu.sync_copy(acc.at[sl], out_hbm.at[c, sl])


_scatter_kernel = functools.partial(
    pl.kernel,
    out_type=jax.ShapeDtypeStruct((NC, N_PAD, D), jnp.float32),
    mesh=_mesh,
    scratch_types=[
        pltpu.VMEM((IB, CHUNK), jnp.int32),
        pltpu.VMEM((IB, CHUNK), jnp.int32),
        pltpu.VMEM((CHUNK, D), jnp.float32),
        pltpu.VMEM((CHUNK, D), jnp.float32),
        pltpu.VMEM_SHARED((N_PAD, D), jnp.float32),
        pltpu.SemaphoreType.DMA,
        pltpu.SemaphoreType.DMA,
        pltpu.SemaphoreType.DMA,
        pltpu.SemaphoreType.DMA,
    ],
)(_scatter_body)


def _prescale_body(x_ref, deg_ref, xs_ref, dinv_ref):
    w = deg_ref[0] + deg_ref[1]
    cnt = jnp.sum(w, axis=1, keepdims=True) * (1.0 / D)
    dinv = lax.rsqrt(cnt + 1.0)
    dinv_ref[...] = dinv
    xs_ref[...] = x_ref[...] * dinv


def _mid_body(p_ref, dinv_ref, w1_ref, b1_ref, w2_ref, t_ref):
    dinv = dinv_ref[...]
    agg = (p_ref[0] + p_ref[1]) * dinv
    h = jnp.dot(agg, w1_ref[...], preferred_element_type=jnp.float32)
    h = jnp.maximum(h + b1_ref[...], 0.0)
    t_ref[...] = jnp.dot(h, w2_ref[...],
                         preferred_element_type=jnp.float32) * dinv


def _final_body(q_ref, dinv_ref, b2_ref, o_ref):
    o_ref[...] = ((q_ref[0, :N] + q_ref[1, :N]) * dinv_ref[:N]
                  + b2_ref[...])


@jax.jit
def kernel(x, edge_index, W1, b1, W2, b2):
    src = edge_index[0].astype(jnp.int32)
    dst = edge_index[1].astype(jnp.int32)
    pad = N + (jnp.arange(EP - E, dtype=jnp.int32) % (N_PAD - N))
    src_p = jnp.concatenate([src, pad]).reshape(NW * K, CHUNK)
    dst_p = jnp.concatenate([dst, pad]).reshape(NW * K, CHUNK)

    x_pad = jnp.concatenate([x, jnp.zeros((N_PAD - N, D), x.dtype)])
    zeros = jnp.zeros((N_PAD, D), jnp.float32)

    deg = _deg_kernel(zeros, dst_p)

    xs, dinv = pl.pallas_call(
        _prescale_body,
        out_shape=[
            jax.ShapeDtypeStruct((N_PAD, D), jnp.float32),
            jax.ShapeDtypeStruct((N_PAD, 1), jnp.float32),
        ],
    )(x_pad, deg)

    p = _scatter_kernel(xs, zeros, src_p, dst_p)

    t = pl.pallas_call(
        _mid_body,
        out_shape=jax.ShapeDtypeStruct((N_PAD, D), jnp.float32),
    )(p, dinv, W1, b1.reshape(1, HID), W2)

    q = _scatter_kernel(t, zeros, src_p, dst_p)

    out = pl.pallas_call(
        _final_body,
        out_shape=jax.ShapeDtypeStruct((N, D), jnp.float32),
    )(q, dinv, b2.reshape(1, D))
    return out


# speedup vs baseline: 28.7965x; 1.1425x over previous
import functools

import jax
import jax.numpy as jnp
from jax import lax
from jax.experimental import pallas as pl
from jax.experimental.pallas import tpu as pltpu
from jax.experimental.pallas import tpu_sc as plsc

N = 10000
D = 128
HID = 256
N_PAD = 10240
NC = 2
NS = 16
NW = NC * NS
CHUNK = 64
E = 320000
K = (-(-E // (NW * CHUNK)) + 7) // 8 * 8
EP = NW * K * CHUNK
RPT = N_PAD // NS
IB = 32

_mesh = plsc.VectorSubcoreMesh(core_axis_name="c", subcore_axis_name="s")


def _deg_body(zeros_hbm, dst_hbm, out_hbm, dst_v, ones_v, acc, sem):
    c = lax.axis_index("c")
    s = lax.axis_index("s")
    wid = c * NS + s
    sl = pl.ds(s * RPT, RPT)

    def fill_ones(i, carry):
        for col in range(D // 16):
            ones_v[i, pl.ds(col * 16, 16)] = jnp.ones((16,), jnp.float32)
        return carry
    lax.fori_loop(0, CHUNK, fill_ones, 0)

    pltpu.sync_copy(zeros_hbm.at[sl], acc.at[sl])
    pltpu.sync_copy(dst_hbm.at[pl.ds(wid * K, K)], dst_v)
    plsc.subcore_barrier()

    FIRE = 8

    def batch(b, carry):
        for u in range(FIRE):
            pltpu.async_copy(ones_v, acc.at[dst_v.at[b * FIRE + u]],
                             sem, add=True)
        for u in range(FIRE):
            pltpu.make_async_copy(ones_v, acc.at[dst_v.at[b * FIRE + u]],
                                  sem).wait()
        return carry
    lax.fori_loop(0, K // FIRE, batch, 0)

    plsc.subcore_barrier()
    pltpu.sync_copy(acc.at[sl], out_hbm.at[c, sl])


_deg_kernel = functools.partial(
    pl.kernel,
    out_type=jax.ShapeDtypeStruct((NC, N_PAD, D), jnp.float32),
    mesh=_mesh,
    scratch_types=[
        pltpu.VMEM((K, CHUNK), jnp.int32),
        pltpu.VMEM((CHUNK, D), jnp.float32),
        pltpu.VMEM_SHARED((N_PAD, D), jnp.float32),
        pltpu.SemaphoreType.DMA,
    ],
)(_deg_body)


def _scatter_body(xs_hbm, zeros_hbm, src_hbm, dst_hbm, out_hbm,
                  src_v, dst_v, r0, r1, r2, r3, acc,
                  g0, g1, g2, g3, s0, s1, s2, s3):
    rows = (r0, r1, r2, r3)
    gs = (g0, g1, g2, g3)
    ss = (s0, s1, s2, s3)
    c = lax.axis_index("c")
    s = lax.axis_index("s")
    wid = c * NS + s
    sl = pl.ds(s * RPT, RPT)

    @pl.when(c == 0)
    def _():
        pltpu.sync_copy(xs_hbm.at[sl], acc.at[sl])

    @pl.when(c == 1)
    def _():
        pltpu.sync_copy(zeros_hbm.at[sl], acc.at[sl])

    plsc.subcore_barrier()

    def stage(g, carry):
        base = wid * K + g * IB
        pltpu.sync_copy(src_hbm.at[pl.ds(base, IB)], src_v)
        pltpu.sync_copy(dst_hbm.at[pl.ds(base, IB)], dst_v)
        for u in range(4):
            pltpu.async_copy(xs_hbm.at[src_v.at[u]], rows[u], gs[u])

        def quad(i, c2):
            j = 4 * i
            for u in range(4):
                pltpu.make_async_copy(xs_hbm.at[src_v.at[j + u]],
                                      rows[u], gs[u]).wait()
                pltpu.async_copy(rows[u], acc.at[dst_v.at[j + u]],
                                 ss[u], add=True)
            for u in range(4):
                @pl.when(j + u + 4 < IB)
                def _(u=u, j=j):
                    pltpu.make_async_copy(rows[u], acc.at[dst_v.at[j + u]],
                                          ss[u]).wait()
                    pltpu.async_copy(xs_hbm.at[src_v.at[j + u + 4]],
                                     rows[u], gs[u])
            return c2
        lax.fori_loop(0, IB // 4, quad, 0)
        for u in range(4):
            pltpu.make_async_copy(rows[u], acc.at[dst_v.at[IB - 4 + u]],
                                  ss[u]).wait()
        return carry
    lax.fori_loop(0, K // IB, stage, 0)

    plsc.subcore_barrier()
    pltpu.sync_copy(acc.at[sl], out_hbm.at[c, sl])


_scatter_kernel = functools.partial(
    pl.kernel,
    out_type=jax.ShapeDtypeStruct((NC, N_PAD, D), jnp.float32),
    mesh=_mesh,
    scratch_types=[
        pltpu.VMEM((IB, CHUNK), jnp.int32),
        pltpu.VMEM((IB, CHUNK), jnp.int32),
        pltpu.VMEM((CHUNK, D), jnp.float32),
        pltpu.VMEM((CHUNK, D), jnp.float32),
        pltpu.VMEM((CHUNK, D), jnp.float32),
        pltpu.VMEM((CHUNK, D), jnp.float32),
        pltpu.VMEM_SHARED((N_PAD, D), jnp.float32),
        pltpu.SemaphoreType.DMA,
        pltpu.SemaphoreType.DMA,
        pltpu.SemaphoreType.DMA,
        pltpu.SemaphoreType.DMA,
        pltpu.SemaphoreType.DMA,
        pltpu.SemaphoreType.DMA,
        pltpu.SemaphoreType.DMA,
        pltpu.SemaphoreType.DMA,
    ],
)(_scatter_body)


def _prescale_body(x_ref, deg_ref, xs_ref, dinv_ref):
    w = deg_ref[0] + deg_ref[1]
    cnt = jnp.sum(w, axis=1, keepdims=True) * (1.0 / D)
    dinv = lax.rsqrt(cnt + 1.0)
    dinv_ref[...] = dinv
    xs_ref[...] = x_ref[...] * dinv


def _mid_body(p_ref, dinv_ref, w1_ref, b1_ref, w2_ref, t_ref):
    dinv = dinv_ref[...]
    agg = (p_ref[0] + p_ref[1]) * dinv
    h = jnp.dot(agg, w1_ref[...], preferred_element_type=jnp.float32)
    h = jnp.maximum(h + b1_ref[...], 0.0)
    t_ref[...] = jnp.dot(h, w2_ref[...],
                         preferred_element_type=jnp.float32) * dinv


def _final_body(q_ref, dinv_ref, b2_ref, o_ref):
    o_ref[...] = ((q_ref[0, :N] + q_ref[1, :N]) * dinv_ref[:N]
                  + b2_ref[...])


@jax.jit
def kernel(x, edge_index, W1, b1, W2, b2):
    src = edge_index[0].astype(jnp.int32)
    dst = edge_index[1].astype(jnp.int32)
    pad = N + (jnp.arange(EP - E, dtype=jnp.int32) % (N_PAD - N))
    src_p = jnp.concatenate([src, pad]).reshape(NW * K, CHUNK)
    dst_p = jnp.concatenate([dst, pad]).reshape(NW * K, CHUNK)

    x_pad = jnp.concatenate([x, jnp.zeros((N_PAD - N, D), x.dtype)])
    zeros = jnp.zeros((N_PAD, D), jnp.float32)

    deg = _deg_kernel(zeros, dst_p)

    xs, dinv = pl.pallas_call(
        _prescale_body,
        out_shape=[
            jax.ShapeDtypeStruct((N_PAD, D), jnp.float32),
            jax.ShapeDtypeStruct((N_PAD, 1), jnp.float32),
        ],
    )(x_pad, deg)

    p = _scatter_kernel(xs, zeros, src_p, dst_p)

    t = pl.pallas_call(
        _mid_body,
        out_shape=jax.ShapeDtypeStruct((N_PAD, D), jnp.float32),
    )(p, dinv, W1, b1.reshape(1, HID), W2)

    q = _scatter_kernel(t, zeros, src_p, dst_p)

    out = pl.pallas_call(
        _final_body,
        out_shape=jax.ShapeDtypeStruct((N, D), jnp.float32),
    )(q, dinv, b2.reshape(1, D))
    return out


# speedup vs baseline: 33.3188x; 1.1570x over previous
import functools

import jax
import jax.numpy as jnp
from jax import lax
from jax.experimental import pallas as pl
from jax.experimental.pallas import tpu as pltpu
from jax.experimental.pallas import tpu_sc as plsc

N = 10000
D = 128
HID = 256
N_PAD = 10240
NC = 2
NS = 16
NW = NC * NS
CHUNK = 64
E = 320000
K = (-(-E // (NW * CHUNK)) + 7) // 8 * 8
EP = NW * K * CHUNK
RPT = N_PAD // NS
IB = 32

_mesh = plsc.VectorSubcoreMesh(core_axis_name="c", subcore_axis_name="s")


def _deg_body(dst_hbm, out_hbm, dst_v, hist, red, shared):
    c = lax.axis_index("c")
    s = lax.axis_index("s")
    wid = c * NS + s
    ones16 = jnp.ones((16,), jnp.float32)

    def zstep(i, carry):
        hist[pl.ds(i * 16, 16)] = jnp.zeros((16,), jnp.float32)
        return carry
    lax.fori_loop(0, N_PAD // 16, zstep, 0)

    pltpu.sync_copy(dst_hbm.at[pl.ds(wid * K, K)], dst_v)

    def step(j, carry):
        for l in range(CHUNK // 16):
            idx = dst_v[j, pl.ds(l * 16, 16)]
            plsc.addupdate_scatter(hist, [idx], ones16)
        return carry
    lax.fori_loop(0, K, step, 0)

    for t in range(NS):
        pltpu.sync_copy(hist.at[pl.ds(t * RPT, RPT)], shared.at[t, s])
    plsc.subcore_barrier()
    pltpu.sync_copy(shared.at[s], red)

    def rstep(i, carry):
        acc16 = jnp.zeros((16,), jnp.float32)
        for t in range(NS):
            acc16 = acc16 + red[t, pl.ds(i * 16, 16)]
        hist[pl.ds(i * 16, 16)] = acc16
        return carry
    lax.fori_loop(0, RPT // 16, rstep, 0)

    pltpu.sync_copy(hist.at[pl.ds(0, RPT)],
                    out_hbm.at[c, pl.ds(s * RPT, RPT)])


_deg_kernel = functools.partial(
    pl.kernel,
    out_type=jax.ShapeDtypeStruct((NC, N_PAD), jnp.float32),
    mesh=_mesh,
    compiler_params=pltpu.CompilerParams(needs_layout_passes=False),
    scratch_types=[
        pltpu.VMEM((K, CHUNK), jnp.int32),
        pltpu.VMEM((N_PAD,), jnp.float32),
        pltpu.VMEM((NS, RPT), jnp.float32),
        pltpu.VMEM_SHARED((NS, NS, RPT), jnp.float32),
    ],
)(_deg_body)


def _scatter_body(xs_hbm, zeros_hbm, src_hbm, dst_hbm, out_hbm,
                  src_v, dst_v, r0, r1, r2, r3, acc,
                  g0, g1, g2, g3, s0, s1, s2, s3):
    rows = (r0, r1, r2, r3)
    gs = (g0, g1, g2, g3)
    ss = (s0, s1, s2, s3)
    c = lax.axis_index("c")
    s = lax.axis_index("s")
    wid = c * NS + s
    sl = pl.ds(s * RPT, RPT)

    @pl.when(c == 0)
    def _():
        pltpu.sync_copy(xs_hbm.at[sl], acc.at[sl])

    @pl.when(c == 1)
    def _():
        pltpu.sync_copy(zeros_hbm.at[sl], acc.at[sl])

    plsc.subcore_barrier()

    def stage(g, carry):
        base = wid * K + g * IB
        pltpu.sync_copy(src_hbm.at[pl.ds(base, IB)], src_v)
        pltpu.sync_copy(dst_hbm.at[pl.ds(base, IB)], dst_v)
        for u in range(4):
            pltpu.async_copy(xs_hbm.at[src_v.at[u]], rows[u], gs[u])

        def quad(i, c2):
            j = 4 * i
            for u in range(4):
                pltpu.make_async_copy(xs_hbm.at[src_v.at[j + u]],
                                      rows[u], gs[u]).wait()
                pltpu.async_copy(rows[u], acc.at[dst_v.at[j + u]],
                                 ss[u], add=True)
            for u in range(4):
                @pl.when(j + u + 4 < IB)
                def _(u=u, j=j):
                    pltpu.make_async_copy(rows[u], acc.at[dst_v.at[j + u]],
                                          ss[u]).wait()
                    pltpu.async_copy(xs_hbm.at[src_v.at[j + u + 4]],
                                     rows[u], gs[u])
            return c2
        lax.fori_loop(0, IB // 4, quad, 0)
        for u in range(4):
            pltpu.make_async_copy(rows[u], acc.at[dst_v.at[IB - 4 + u]],
                                  ss[u]).wait()
        return carry
    lax.fori_loop(0, K // IB, stage, 0)

    plsc.subcore_barrier()
    pltpu.sync_copy(acc.at[sl], out_hbm.at[c, sl])


_scatter_kernel = functools.partial(
    pl.kernel,
    out_type=jax.ShapeDtypeStruct((NC, N_PAD, D), jnp.float32),
    mesh=_mesh,
    scratch_types=[
        pltpu.VMEM((IB, CHUNK), jnp.int32),
        pltpu.VMEM((IB, CHUNK), jnp.int32),
        pltpu.VMEM((CHUNK, D), jnp.float32),
        pltpu.VMEM((CHUNK, D), jnp.float32),
        pltpu.VMEM((CHUNK, D), jnp.float32),
        pltpu.VMEM((CHUNK, D), jnp.float32),
        pltpu.VMEM_SHARED((N_PAD, D), jnp.float32),
        pltpu.SemaphoreType.DMA,
        pltpu.SemaphoreType.DMA,
        pltpu.SemaphoreType.DMA,
        pltpu.SemaphoreType.DMA,
        pltpu.SemaphoreType.DMA,
        pltpu.SemaphoreType.DMA,
        pltpu.SemaphoreType.DMA,
        pltpu.SemaphoreType.DMA,
    ],
)(_scatter_body)


def _prescale_body(x_ref, deg_ref, xs_ref, dinv_ref):
    cnt = deg_ref[0] + deg_ref[1]
    dinv = lax.rsqrt(cnt + 1.0)
    dinv_ref[...] = dinv
    xs_ref[...] = x_ref[...] * dinv


def _mid_body(p_ref, dinv_ref, w1_ref, b1_ref, w2_ref, t_ref):
    dinv = dinv_ref[...]
    agg = (p_ref[0] + p_ref[1]) * dinv
    h = jnp.dot(agg, w1_ref[...], preferred_element_type=jnp.float32)
    h = jnp.maximum(h + b1_ref[...], 0.0)
    t_ref[...] = jnp.dot(h, w2_ref[...],
                         preferred_element_type=jnp.float32) * dinv


def _final_body(q_ref, dinv_ref, b2_ref, o_ref):
    o_ref[...] = ((q_ref[0, :N] + q_ref[1, :N]) * dinv_ref[:N]
                  + b2_ref[...])


@jax.jit
def kernel(x, edge_index, W1, b1, W2, b2):
    src = edge_index[0].astype(jnp.int32)
    dst = edge_index[1].astype(jnp.int32)
    pad = N + (jnp.arange(EP - E, dtype=jnp.int32) % (N_PAD - N))
    src_p = jnp.concatenate([src, pad]).reshape(NW * K, CHUNK)
    dst_p = jnp.concatenate([dst, pad]).reshape(NW * K, CHUNK)

    x_pad = jnp.concatenate([x, jnp.zeros((N_PAD - N, D), x.dtype)])
    zeros = jnp.zeros((N_PAD, D), jnp.float32)

    deg = _deg_kernel(dst_p).reshape(NC, N_PAD, 1)

    xs, dinv = pl.pallas_call(
        _prescale_body,
        out_shape=[
            jax.ShapeDtypeStruct((N_PAD, D), jnp.float32),
            jax.ShapeDtypeStruct((N_PAD, 1), jnp.float32),
        ],
    )(x_pad, deg)

    p = _scatter_kernel(xs, zeros, src_p, dst_p)

    t = pl.pallas_call(
        _mid_body,
        out_shape=jax.ShapeDtypeStruct((N_PAD, D), jnp.float32),
    )(p, dinv, W1, b1.reshape(1, HID), W2)

    q = _scatter_kernel(t, zeros, src_p, dst_p)

    out = pl.pallas_call(
        _final_body,
        out_shape=jax.ShapeDtypeStruct((N, D), jnp.float32),
    )(q, dinv, b2.reshape(1, D))
    return out


# speedup vs baseline: 33.7102x; 1.0117x over previous
import functools

import jax
import jax.numpy as jnp
from jax import lax
from jax.experimental import pallas as pl
from jax.experimental.pallas import tpu as pltpu
from jax.experimental.pallas import tpu_sc as plsc

N = 10000
D = 128
HID = 256
N_PAD = 10240
NC = 2
NS = 16
NW = NC * NS
CHUNK = 64
E = 320000
K = (-(-E // (NW * CHUNK)) + 7) // 8 * 8
EP = NW * K * CHUNK
RPT = N_PAD // NS
IB = 40

_mesh = plsc.VectorSubcoreMesh(core_axis_name="c", subcore_axis_name="s")


def _deg_body(dst_hbm, out_hbm, dst_v, hist, red, shared):
    c = lax.axis_index("c")
    s = lax.axis_index("s")
    wid = c * NS + s
    ones16 = jnp.ones((16,), jnp.float32)

    def zstep(i, carry):
        hist[pl.ds(i * 16, 16)] = jnp.zeros((16,), jnp.float32)
        return carry
    lax.fori_loop(0, N_PAD // 16, zstep, 0)

    pltpu.sync_copy(dst_hbm.at[pl.ds(wid * K, K)], dst_v)

    def step(j, carry):
        for l in range(CHUNK // 16):
            idx = dst_v[j, pl.ds(l * 16, 16)]
            plsc.addupdate_scatter(hist, [idx], ones16)
        return carry
    lax.fori_loop(0, K, step, 0)

    for t in range(NS):
        pltpu.sync_copy(hist.at[pl.ds(t * RPT, RPT)], shared.at[t, s])
    plsc.subcore_barrier()
    pltpu.sync_copy(shared.at[s], red)

    def rstep(i, carry):
        acc16 = jnp.zeros((16,), jnp.float32)
        for t in range(NS):
            acc16 = acc16 + red[t, pl.ds(i * 16, 16)]
        hist[pl.ds(i * 16, 16)] = acc16
        return carry
    lax.fori_loop(0, RPT // 16, rstep, 0)

    pltpu.sync_copy(hist.at[pl.ds(0, RPT)],
                    out_hbm.at[c, pl.ds(s * RPT, RPT)])


_deg_kernel = functools.partial(
    pl.kernel,
    out_type=jax.ShapeDtypeStruct((NC, N_PAD), jnp.float32),
    mesh=_mesh,
    compiler_params=pltpu.CompilerParams(needs_layout_passes=False),
    scratch_types=[
        pltpu.VMEM((K, CHUNK), jnp.int32),
        pltpu.VMEM((N_PAD,), jnp.float32),
        pltpu.VMEM((NS, RPT), jnp.float32),
        pltpu.VMEM_SHARED((NS, NS, RPT), jnp.float32),
    ],
)(_deg_body)


def _scatter_body(xs_hbm, zeros_hbm, src_hbm, dst_hbm, out_hbm,
                  src_v, dst_v, r0, r1, r2, r3, acc,
                  g0, g1, g2, g3, s0, s1, s2, s3):
    rows = (r0, r1, r2, r3)
    gs = (g0, g1, g2, g3)
    ss = (s0, s1, s2, s3)
    c = lax.axis_index("c")
    s = lax.axis_index("s")
    wid = c * NS + s
    sl = pl.ds(s * RPT, RPT)

    @pl.when(c == 0)
    def _():
        pltpu.sync_copy(xs_hbm.at[sl], acc.at[sl])

    @pl.when(c == 1)
    def _():
        pltpu.sync_copy(zeros_hbm.at[sl], acc.at[sl])

    plsc.subcore_barrier()

    def stage(g, carry):
        base = wid * K + g * IB
        pltpu.sync_copy(src_hbm.at[pl.ds(base, IB)], src_v)
        pltpu.sync_copy(dst_hbm.at[pl.ds(base, IB)], dst_v)
        for u in range(4):
            pltpu.async_copy(xs_hbm.at[src_v.at[u]], rows[u], gs[u])

        def quad(i, c2):
            j = 4 * i
            for u in range(4):
                pltpu.make_async_copy(xs_hbm.at[src_v.at[j + u]],
                                      rows[u], gs[u]).wait()
                pltpu.async_copy(rows[u], acc.at[dst_v.at[j + u]],
                                 ss[u], add=True)
            for u in range(4):
                @pl.when(j + u + 4 < IB)
                def _(u=u, j=j):
                    pltpu.make_async_copy(rows[u], acc.at[dst_v.at[j + u]],
                                          ss[u]).wait()
                    pltpu.async_copy(xs_hbm.at[src_v.at[j + u + 4]],
                                     rows[u], gs[u])
            return c2
        lax.fori_loop(0, IB // 4, quad, 0)
        for u in range(4):
            pltpu.make_async_copy(rows[u], acc.at[dst_v.at[IB - 4 + u]],
                                  ss[u]).wait()
        return carry
    lax.fori_loop(0, K // IB, stage, 0)

    plsc.subcore_barrier()
    pltpu.sync_copy(acc.at[sl], out_hbm.at[c, sl])


_scatter_kernel = functools.partial(
    pl.kernel,
    out_type=jax.ShapeDtypeStruct((NC, N_PAD, D), jnp.float32),
    mesh=_mesh,
    scratch_types=[
        pltpu.VMEM((IB, CHUNK), jnp.int32),
        pltpu.VMEM((IB, CHUNK), jnp.int32),
        pltpu.VMEM((CHUNK, D), jnp.float32),
        pltpu.VMEM((CHUNK, D), jnp.float32),
        pltpu.VMEM((CHUNK, D), jnp.float32),
        pltpu.VMEM((CHUNK, D), jnp.float32),
        pltpu.VMEM_SHARED((N_PAD, D), jnp.float32),
        pltpu.SemaphoreType.DMA,
        pltpu.SemaphoreType.DMA,
        pltpu.SemaphoreType.DMA,
        pltpu.SemaphoreType.DMA,
        pltpu.SemaphoreType.DMA,
        pltpu.SemaphoreType.DMA,
        pltpu.SemaphoreType.DMA,
        pltpu.SemaphoreType.DMA,
    ],
)(_scatter_body)


def _prescale_body(x_ref, deg_ref, xs_ref, dinv_ref):
    cnt = deg_ref[0] + deg_ref[1]
    dinv = lax.rsqrt(cnt + 1.0)
    dinv_ref[...] = dinv
    xs_ref[...] = x_ref[...] * dinv


def _mid_body(p_ref, dinv_ref, w1_ref, b1_ref, w2_ref, t_ref):
    dinv = dinv_ref[...]
    agg = (p_ref[0] + p_ref[1]) * dinv
    h = jnp.dot(agg, w1_ref[...], preferred_element_type=jnp.float32)
    h = jnp.maximum(h + b1_ref[...], 0.0)
    t_ref[...] = jnp.dot(h, w2_ref[...],
                         preferred_element_type=jnp.float32) * dinv


def _final_body(q_ref, dinv_ref, b2_ref, o_ref):
    o_ref[...] = ((q_ref[0, :N] + q_ref[1, :N]) * dinv_ref[:N]
                  + b2_ref[...])


@jax.jit
def kernel(x, edge_index, W1, b1, W2, b2):
    src = edge_index[0].astype(jnp.int32)
    dst = edge_index[1].astype(jnp.int32)
    pad = N + (jnp.arange(EP - E, dtype=jnp.int32) % (N_PAD - N))
    src_p = jnp.concatenate([src, pad]).reshape(NW * K, CHUNK)
    dst_p = jnp.concatenate([dst, pad]).reshape(NW * K, CHUNK)

    x_pad = jnp.concatenate([x, jnp.zeros((N_PAD - N, D), x.dtype)])
    zeros = jnp.zeros((N_PAD, D), jnp.float32)

    deg = _deg_kernel(dst_p).reshape(NC, N_PAD, 1)

    xs, dinv = pl.pallas_call(
        _prescale_body,
        out_shape=[
            jax.ShapeDtypeStruct((N_PAD, D), jnp.float32),
            jax.ShapeDtypeStruct((N_PAD, 1), jnp.float32),
        ],
    )(x_pad, deg)

    p = _scatter_kernel(xs, zeros, src_p, dst_p)

    t = pl.pallas_call(
        _mid_body,
        out_shape=jax.ShapeDtypeStruct((N_PAD, D), jnp.float32),
    )(p, dinv, W1, b1.reshape(1, HID), W2)

    q = _scatter_kernel(t, zeros, src_p, dst_p)

    out = pl.pallas_call(
        _final_body,
        out_shape=jax.ShapeDtypeStruct((N, D), jnp.float32),
    )(q, dinv, b2.reshape(1, D))
    return out
